# baseline (speedup 1.0000x reference)
import functools

import jax
import jax.numpy as jnp
from jax import lax
from jax.experimental import pallas as pl
from jax.experimental.pallas import tpu as pltpu
from jax.experimental.pallas import tpu_sc as plsc

Q = 200
N = 50000
NTOT = Q * N
NCLS = 19
NON_OBJECT_WEIGHT = 0.1

NC = 2
NS = 16
NW = NC * NS

K = 4096
XLO = -8.0
XHI = 8.0
CHUNK = 10000
NCHUNKS = NTOT // CHUNK
ROWS_PER_STEP = 8
GRID1 = Q // ROWS_PER_STEP


def _pass1_body(x_ref, t_ref, rowstats_ref, scal_ref):
    i = pl.program_id(0)
    x = x_ref[...]
    t = t_ref[...]
    p = 1.0 / (1.0 + jnp.exp(-x))
    bce = jnp.maximum(x, 0.0) - x * t - jnp.log(jnp.maximum(p, 1.0 - p))
    bce_s = jnp.sum(bce)
    s_pt = jnp.sum(p * t, axis=1)
    s_p = jnp.sum(p, axis=1)
    s_t = jnp.sum(t, axis=1)
    col = lax.broadcasted_iota(jnp.int32, x.shape, 1).astype(jnp.float32)
    pm = p > 0.0
    gm = t > 0.0
    big = jnp.float32(N + 1)
    pmax_s = jnp.max(jnp.where(pm, col, -1.0))
    pmin_s = jnp.min(jnp.where(pm, col, big))
    gmax_s = jnp.max(jnp.where(gm, col, -1.0))
    gmin_s = jnp.min(jnp.where(gm, col, big))
    s1_s = jnp.sum(jnp.where(t > 0.5, 1.0 - p, 0.0))

    lane_r = lax.broadcasted_iota(jnp.int32, (ROWS_PER_STEP, 128), 1)
    rs = jnp.where(lane_r == 0, s_pt[:, None],
                   jnp.where(lane_r == 1, s_p[:, None],
                             jnp.where(lane_r == 2, s_t[:, None], 0.0)))
    rowstats_ref[...] = rs

    lane = lax.broadcasted_iota(jnp.int32, (1, 128), 1)
    v = jnp.where(lane == 0, bce_s,
        jnp.where(lane == 1, s1_s,
        jnp.where(lane == 2, pmax_s,
        jnp.where(lane == 3, pmin_s,
        jnp.where(lane == 4, gmax_s,
        jnp.where(lane == 5, gmin_s, 0.0))))))

    @pl.when(i == 0)
    def _init():
        scal_ref[...] = v

    @pl.when(i != 0)
    def _acc():
        old = scal_ref[...]
        upd = jnp.where(lane < 2, old + v,
              jnp.where((lane == 2) | (lane == 4), jnp.maximum(old, v),
              jnp.where((lane == 3) | (lane == 5), jnp.minimum(old, v), old)))
        scal_ref[...] = upd


def _run_pass1(pred_mask, tgt_mask):
    return pl.pallas_call(
        _pass1_body,
        grid=(GRID1,),
        in_specs=[
            pl.BlockSpec((ROWS_PER_STEP, N), lambda i: (i, 0)),
            pl.BlockSpec((ROWS_PER_STEP, N), lambda i: (i, 0)),
        ],
        out_specs=[
            pl.BlockSpec((ROWS_PER_STEP, 128), lambda i: (i, 0)),
            pl.BlockSpec((1, 128), lambda i: (0, 0)),
        ],
        out_shape=[
            jax.ShapeDtypeStruct((Q, 128), jnp.float32),
            jax.ShapeDtypeStruct((1, 128), jnp.float32),
        ],
    )(pred_mask, tgt_mask)


def _sc_hist_body(x_hbm, t_hbm, out_hbm,
                  xbuf0, tbuf0, xbuf1, tbuf1, hist,
                  semx0, semt0, semx1, semt1):
    wid = lax.axis_index("s") * NC + lax.axis_index("c")
    zeros16 = jnp.zeros((16,), jnp.float32)
    ones16 = jnp.ones((16,), jnp.float32)
    lane16 = lax.iota(jnp.int32, 16)
    scale = jnp.float32(K / (XHI - XLO))
    off0 = jnp.float32(-XLO * K / (XHI - XLO))
    xbufs = (xbuf0, xbuf1)
    tbufs = (tbuf0, tbuf1)
    semxs = (semx0, semx1)
    semts = (semt0, semt1)

    @pl.loop(0, (NS * K) // 16)
    def _zero(z):
        hist[pl.ds(z * 16, 16)] = zeros16

    nchunks = jnp.where(wid < (NCHUNKS % NW), NCHUNKS // NW + 1, NCHUNKS // NW)

    def issue(ci, b):
        base = (wid + ci * NW) * CHUNK
        pltpu.async_copy(x_hbm.at[pl.ds(base, CHUNK)], xbufs[b], semxs[b])
        pltpu.async_copy(t_hbm.at[pl.ds(base, CHUNK)], tbufs[b], semts[b])

    issue(0, 0)
    npairs = (NCHUNKS // NW + 2) // 2

    @pl.loop(0, npairs)
    def _pair(pi):
        for b in (0, 1):
            ci = pi * 2 + b

            @pl.when(ci < nchunks)
            def _one():
                pltpu.make_async_copy(
                    x_hbm.at[pl.ds(0, CHUNK)], xbufs[b], semxs[b]).wait()
                pltpu.make_async_copy(
                    t_hbm.at[pl.ds(0, CHUNK)], tbufs[b], semts[b]).wait()

                @pl.when(ci + 1 < nchunks)
                def _prefetch():
                    issue(ci + 1, 1 - b)

                xbuf = xbufs[b]
                tbuf = tbufs[b]

                @pl.loop(0, CHUNK // 128)
                def _vec(j8):
                    base8 = j8 * 128
                    for u in range(8):
                        xv = xbuf[pl.ds(base8 + u * 16, 16)]
                        tv = tbuf[pl.ds(base8 + u * 16, 16)]
                        binf = xv * scale + off0
                        bin_ = jnp.clip(binf.astype(jnp.int32), 0, K - 1)
                        idx = bin_ * 16 + lane16
                        plsc.addupdate_scatter(hist, [idx], ones16,
                                               mask=tv < 0.5)

    pltpu.sync_copy(hist, out_hbm.at[pl.ds(wid * NS * K, NS * K)])


def _run_sc_hist(x_flat, t_flat):
    mesh = plsc.VectorSubcoreMesh(core_axis_name="c", subcore_axis_name="s")
    k = pl.kernel(
        _sc_hist_body,
        out_type=jax.ShapeDtypeStruct((NW * NS * K,), jnp.float32),
        mesh=mesh,
        scratch_types=[
            pltpu.VMEM((CHUNK,), jnp.float32),
            pltpu.VMEM((CHUNK,), jnp.float32),
            pltpu.VMEM((CHUNK,), jnp.float32),
            pltpu.VMEM((CHUNK,), jnp.float32),
            pltpu.VMEM((NS * K,), jnp.float32),
            pltpu.SemaphoreType.DMA,
            pltpu.SemaphoreType.DMA,
            pltpu.SemaphoreType.DMA,
            pltpu.SemaphoreType.DMA,
        ],
        compiler_params=pltpu.CompilerParams(needs_layout_passes=False),
    )
    return k(x_flat, t_flat)


def _combine_body(hist_ref, rowstats_ref, scal_ref, pl_ref, gt_ref, lw_ref,
                  out_ref):
    scal = scal_ref[...]
    lane = lax.broadcasted_iota(jnp.int32, (1, 128), 1)

    def pick(j):
        return jnp.sum(jnp.where(lane == j, scal, 0.0))

    bce_sum = pick(0)
    s1 = pick(1)
    pmax = pick(2)
    pmin = pick(3)
    gmax = pick(4)
    gmin = pick(5)

    rs = rowstats_ref[...]
    s_pt = rs[:, 0:1]
    s_p = rs[:, 1:2]
    s_t = rs[:, 2:3]

    plog = pl_ref[...]
    g = gt_ref[...]
    m = jnp.max(plog, axis=1, keepdims=True)
    lse = m + jnp.log(jnp.sum(jnp.exp(plog - m), axis=1, keepdims=True))
    cls_iota = lax.broadcasted_iota(jnp.int32, (Q, NCLS), 1)
    onehot = (cls_iota == g).astype(jnp.float32)
    picked = jnp.sum(plog * onehot, axis=1, keepdims=True)
    ce = lse - picked
    w = jnp.where(g == NCLS - 1, NON_OBJECT_WEIGHT, 1.0).astype(jnp.float32)
    loss_class = jnp.sum(w * ce) / jnp.sum(w)

    n_f = jnp.float32(NTOT)
    loss_bce = bce_sum / n_f
    loss_dice = jnp.sum(1.0 - (2.0 * s_pt + 1.0) / (s_p + s_t + 1.0)) / Q
    eps = jnp.float32(1e-6)
    union = s_p + s_t - s_pt
    iou = s_pt / (union + eps)
    encl = (pmax - pmin) * (gmax - gmin)
    giou = iou - (encl - union) / (encl + eps)
    loss_giou = jnp.sum(1.0 - giou) / Q

    G = jnp.sum(s_t)
    n0 = n_f - G
    h3 = hist_ref[...]
    h512 = jnp.sum(h3, axis=0)
    cc = lax.broadcasted_iota(jnp.int32, (128, 8), 0)
    ss = lax.broadcasted_iota(jnp.int32, (128, 8), 1)
    gather16 = ((cc >> 4) == ss).astype(jnp.float32)
    hb = lax.dot_general(h512, gather16, (((1,), (0,)), ((), ())),
                         preferred_element_type=jnp.float32)
    ui = lax.broadcasted_iota(jnp.int32, (8, 8), 0)
    uj = lax.broadcasted_iota(jnp.int32, (8, 8), 1)
    upper8 = (ui <= uj).astype(jnp.float32)
    c1 = lax.dot_general(hb, upper8, (((1,), (0,)), ((), ())),
                         preferred_element_type=jnp.float32)
    rt = c1[:, 7:8]
    si = lax.broadcasted_iota(jnp.int32, (512, 512), 0)
    sj = lax.broadcasted_iota(jnp.int32, (512, 512), 1)
    strictl = (sj < si).astype(jnp.float32)
    off = lax.dot_general(strictl, rt, (((1,), (0,)), ((), ())),
                          preferred_element_type=jnp.float32)
    S = c1 + off
    a = n0 - S
    b = a + hb
    kr = lax.broadcasted_iota(jnp.int32, (512, 8), 0).astype(jnp.float32)
    kc = lax.broadcasted_iota(jnp.int32, (512, 8), 1).astype(jnp.float32)
    kidx = kr * 8.0 + kc
    center = XLO + (kidx + 0.5) * ((XHI - XLO) / K)
    ebar = 1.0 + 1.0 / (1.0 + jnp.exp(-center))
    terms = ebar * G * hb / ((G + a) * (G + b))
    loss_lovasz = s1 / n_f + jnp.sum(terms)

    lw = lw_ref[...]
    lane8 = lax.broadcasted_iota(jnp.int32, (1, 8), 1)
    tv = jnp.where(lane8 == 0, loss_class,
         jnp.where(lane8 == 1, loss_bce,
         jnp.where(lane8 == 2, loss_dice,
         jnp.where(lane8 == 3, loss_giou,
         jnp.where(lane8 == 4, loss_lovasz, 0.0)))))
    out_ref[...] = jnp.sum(lw * tv).reshape(1, 1)


def _run_combine(hist, rowstats, scal, pred_labels, gt_labels, loss_weight):
    hist3 = hist.reshape(NW, 512, 128)
    gt2 = gt_labels.astype(jnp.int32).reshape(Q, 1)
    lw8 = jnp.zeros((1, 8), jnp.float32).at[0, :5].set(loss_weight)
    return pl.pallas_call(
        _combine_body,
        out_shape=jax.ShapeDtypeStruct((1, 1), jnp.float32),
    )(hist3, rowstats, scal, pred_labels, gt2, lw8)


def kernel(pred_labels, pred_mask, tgt_mask, gt_labels, loss_weight):
    hist = _run_sc_hist(pred_mask.reshape(NTOT), tgt_mask.reshape(NTOT))
    rowstats, scal = _run_pass1(pred_mask, tgt_mask)
    out = _run_combine(hist, rowstats, scal, pred_labels, gt_labels,
                       loss_weight)
    return out.reshape(())


# speedup vs baseline: 1.0066x; 1.0066x over previous
import functools

import jax
import jax.numpy as jnp
from jax import lax
from jax.experimental import pallas as pl
from jax.experimental.pallas import tpu as pltpu
from jax.experimental.pallas import tpu_sc as plsc

Q = 200
N = 50000
NTOT = Q * N
NCLS = 19
NON_OBJECT_WEIGHT = 0.1

NC = 2
NS = 16
NW = NC * NS

K = 2048
XLO = -8.0
XHI = 8.0
CHUNK = 10000
NCHUNKS = NTOT // CHUNK
ROWS_PER_STEP = 8
GRID1 = Q // ROWS_PER_STEP


def _pass1_body(x_ref, t_ref, rowstats_ref, scal_ref):
    i = pl.program_id(0)
    x = x_ref[...]
    t = t_ref[...]
    p = 1.0 / (1.0 + jnp.exp(-x))
    bce = jnp.maximum(x, 0.0) - x * t - jnp.log(jnp.maximum(p, 1.0 - p))
    bce_s = jnp.sum(bce)
    s_pt = jnp.sum(p * t, axis=1)
    s_p = jnp.sum(p, axis=1)
    s_t = jnp.sum(t, axis=1)
    col = lax.broadcasted_iota(jnp.int32, x.shape, 1).astype(jnp.float32)
    pm = p > 0.0
    gm = t > 0.0
    big = jnp.float32(N + 1)
    pmax_s = jnp.max(jnp.where(pm, col, -1.0))
    pmin_s = jnp.min(jnp.where(pm, col, big))
    gmax_s = jnp.max(jnp.where(gm, col, -1.0))
    gmin_s = jnp.min(jnp.where(gm, col, big))
    s1_s = jnp.sum(jnp.where(t > 0.5, 1.0 - p, 0.0))

    lane_r = lax.broadcasted_iota(jnp.int32, (ROWS_PER_STEP, 128), 1)
    rs = jnp.where(lane_r == 0, s_pt[:, None],
                   jnp.where(lane_r == 1, s_p[:, None],
                             jnp.where(lane_r == 2, s_t[:, None], 0.0)))
    rowstats_ref[...] = rs

    lane = lax.broadcasted_iota(jnp.int32, (1, 128), 1)
    v = jnp.where(lane == 0, bce_s,
        jnp.where(lane == 1, s1_s,
        jnp.where(lane == 2, pmax_s,
        jnp.where(lane == 3, pmin_s,
        jnp.where(lane == 4, gmax_s,
        jnp.where(lane == 5, gmin_s, 0.0))))))

    @pl.when(i == 0)
    def _init():
        scal_ref[...] = v

    @pl.when(i != 0)
    def _acc():
        old = scal_ref[...]
        upd = jnp.where(lane < 2, old + v,
              jnp.where((lane == 2) | (lane == 4), jnp.maximum(old, v),
              jnp.where((lane == 3) | (lane == 5), jnp.minimum(old, v), old)))
        scal_ref[...] = upd


def _run_pass1(pred_mask, tgt_mask):
    return pl.pallas_call(
        _pass1_body,
        grid=(GRID1,),
        in_specs=[
            pl.BlockSpec((ROWS_PER_STEP, N), lambda i: (i, 0)),
            pl.BlockSpec((ROWS_PER_STEP, N), lambda i: (i, 0)),
        ],
        out_specs=[
            pl.BlockSpec((ROWS_PER_STEP, 128), lambda i: (i, 0)),
            pl.BlockSpec((1, 128), lambda i: (0, 0)),
        ],
        out_shape=[
            jax.ShapeDtypeStruct((Q, 128), jnp.float32),
            jax.ShapeDtypeStruct((1, 128), jnp.float32),
        ],
    )(pred_mask, tgt_mask)


def _sc_hist_body(x_hbm, t_hbm, out_hbm,
                  xbuf0, tbuf0, xbuf1, tbuf1, hist, histb,
                  semx0, semt0, semx1, semt1):
    wid = lax.axis_index("s") * NC + lax.axis_index("c")
    zeros16 = jnp.zeros((16,), jnp.float32)
    ones16 = jnp.ones((16,), jnp.float32)
    lane16 = lax.iota(jnp.int32, 16)
    scale = jnp.float32(K / (XHI - XLO))
    off0 = jnp.float32(-XLO * K / (XHI - XLO))
    xbufs = (xbuf0, xbuf1)
    tbufs = (tbuf0, tbuf1)
    semxs = (semx0, semx1)
    semts = (semt0, semt1)

    @pl.loop(0, (NS * K) // 16)
    def _zero(z):
        hist[pl.ds(z * 16, 16)] = zeros16
        histb[pl.ds(z * 16, 16)] = zeros16

    nchunks = jnp.where(wid < (NCHUNKS % NW), NCHUNKS // NW + 1, NCHUNKS // NW)

    def issue(ci, b):
        base = (wid + ci * NW) * CHUNK
        pltpu.async_copy(x_hbm.at[pl.ds(base, CHUNK)], xbufs[b], semxs[b])
        pltpu.async_copy(t_hbm.at[pl.ds(base, CHUNK)], tbufs[b], semts[b])

    issue(0, 0)
    npairs = (NCHUNKS // NW + 2) // 2

    @pl.loop(0, npairs)
    def _pair(pi):
        for b in (0, 1):
            ci = pi * 2 + b

            @pl.when(ci < nchunks)
            def _one():
                pltpu.make_async_copy(
                    x_hbm.at[pl.ds(0, CHUNK)], xbufs[b], semxs[b]).wait()
                pltpu.make_async_copy(
                    t_hbm.at[pl.ds(0, CHUNK)], tbufs[b], semts[b]).wait()

                @pl.when(ci + 1 < nchunks)
                def _prefetch():
                    issue(ci + 1, 1 - b)

                xbuf = xbufs[b]
                tbuf = tbufs[b]

                @pl.loop(0, CHUNK // 128)
                def _vec(j8):
                    base8 = j8 * 128
                    for u in range(8):
                        xv = xbuf[pl.ds(base8 + u * 16, 16)]
                        tv = tbuf[pl.ds(base8 + u * 16, 16)]
                        binf = xv * scale + off0
                        bin_ = jnp.clip(binf.astype(jnp.int32), 0, K - 1)
                        idx = bin_ * 16 + lane16
                        plsc.addupdate_scatter(hist if u % 2 == 0 else histb,
                                               [idx], ones16,
                                               mask=tv < 0.5)

    @pl.loop(0, (NS * K) // 16)
    def _merge(z):
        hist[pl.ds(z * 16, 16)] += histb[pl.ds(z * 16, 16)]

    pltpu.sync_copy(hist, out_hbm.at[pl.ds(wid * NS * K, NS * K)])


def _run_sc_hist(x_flat, t_flat):
    mesh = plsc.VectorSubcoreMesh(core_axis_name="c", subcore_axis_name="s")
    k = pl.kernel(
        _sc_hist_body,
        out_type=jax.ShapeDtypeStruct((NW * NS * K,), jnp.float32),
        mesh=mesh,
        scratch_types=[
            pltpu.VMEM((CHUNK,), jnp.float32),
            pltpu.VMEM((CHUNK,), jnp.float32),
            pltpu.VMEM((CHUNK,), jnp.float32),
            pltpu.VMEM((CHUNK,), jnp.float32),
            pltpu.VMEM((NS * K,), jnp.float32),
            pltpu.VMEM((NS * K,), jnp.float32),
            pltpu.SemaphoreType.DMA,
            pltpu.SemaphoreType.DMA,
            pltpu.SemaphoreType.DMA,
            pltpu.SemaphoreType.DMA,
        ],
        compiler_params=pltpu.CompilerParams(needs_layout_passes=False),
    )
    return k(x_flat, t_flat)


def _combine_body(hist_ref, rowstats_ref, scal_ref, pl_ref, gt_ref, lw_ref,
                  out_ref):
    scal = scal_ref[...]
    lane = lax.broadcasted_iota(jnp.int32, (1, 128), 1)

    def pick(j):
        return jnp.sum(jnp.where(lane == j, scal, 0.0))

    bce_sum = pick(0)
    s1 = pick(1)
    pmax = pick(2)
    pmin = pick(3)
    gmax = pick(4)
    gmin = pick(5)

    rs = rowstats_ref[...]
    s_pt = rs[:, 0:1]
    s_p = rs[:, 1:2]
    s_t = rs[:, 2:3]

    plog = pl_ref[...]
    g = gt_ref[...]
    m = jnp.max(plog, axis=1, keepdims=True)
    lse = m + jnp.log(jnp.sum(jnp.exp(plog - m), axis=1, keepdims=True))
    cls_iota = lax.broadcasted_iota(jnp.int32, (Q, NCLS), 1)
    onehot = (cls_iota == g).astype(jnp.float32)
    picked = jnp.sum(plog * onehot, axis=1, keepdims=True)
    ce = lse - picked
    w = jnp.where(g == NCLS - 1, NON_OBJECT_WEIGHT, 1.0).astype(jnp.float32)
    loss_class = jnp.sum(w * ce) / jnp.sum(w)

    n_f = jnp.float32(NTOT)
    loss_bce = bce_sum / n_f
    loss_dice = jnp.sum(1.0 - (2.0 * s_pt + 1.0) / (s_p + s_t + 1.0)) / Q
    eps = jnp.float32(1e-6)
    union = s_p + s_t - s_pt
    iou = s_pt / (union + eps)
    encl = (pmax - pmin) * (gmax - gmin)
    giou = iou - (encl - union) / (encl + eps)
    loss_giou = jnp.sum(1.0 - giou) / Q

    G = jnp.sum(s_t)
    n0 = n_f - G
    nrows = (K * 16) // 128
    h3 = hist_ref[...]
    h512 = jnp.sum(h3, axis=0)
    cc = lax.broadcasted_iota(jnp.int32, (128, 8), 0)
    ss = lax.broadcasted_iota(jnp.int32, (128, 8), 1)
    gather16 = ((cc >> 4) == ss).astype(jnp.float32)
    hb = lax.dot_general(h512, gather16, (((1,), (0,)), ((), ())),
                         preferred_element_type=jnp.float32)
    ui = lax.broadcasted_iota(jnp.int32, (8, 8), 0)
    uj = lax.broadcasted_iota(jnp.int32, (8, 8), 1)
    upper8 = (ui <= uj).astype(jnp.float32)
    c1 = lax.dot_general(hb, upper8, (((1,), (0,)), ((), ())),
                         preferred_element_type=jnp.float32)
    rt = c1[:, 7:8]
    si = lax.broadcasted_iota(jnp.int32, (nrows, nrows), 0)
    sj = lax.broadcasted_iota(jnp.int32, (nrows, nrows), 1)
    strictl = (sj < si).astype(jnp.float32)
    off = lax.dot_general(strictl, rt, (((1,), (0,)), ((), ())),
                          preferred_element_type=jnp.float32)
    S = c1 + off
    a = n0 - S
    b = a + hb
    kr = lax.broadcasted_iota(jnp.int32, (nrows, 8), 0).astype(jnp.float32)
    kc = lax.broadcasted_iota(jnp.int32, (nrows, 8), 1).astype(jnp.float32)
    kidx = kr * 8.0 + kc
    center = XLO + (kidx + 0.5) * ((XHI - XLO) / K)
    ebar = 1.0 + 1.0 / (1.0 + jnp.exp(-center))
    terms = ebar * G * hb / ((G + a) * (G + b))
    loss_lovasz = s1 / n_f + jnp.sum(terms)

    lw = lw_ref[...]
    lane8 = lax.broadcasted_iota(jnp.int32, (1, 8), 1)
    tv = jnp.where(lane8 == 0, loss_class,
         jnp.where(lane8 == 1, loss_bce,
         jnp.where(lane8 == 2, loss_dice,
         jnp.where(lane8 == 3, loss_giou,
         jnp.where(lane8 == 4, loss_lovasz, 0.0)))))
    out_ref[...] = jnp.sum(lw * tv).reshape(1, 1)


def _run_combine(hist, rowstats, scal, pred_labels, gt_labels, loss_weight):
    hist3 = hist.reshape(NW, (K * 16) // 128, 128)
    gt2 = gt_labels.astype(jnp.int32).reshape(Q, 1)
    lw8 = jnp.zeros((1, 8), jnp.float32).at[0, :5].set(loss_weight)
    return pl.pallas_call(
        _combine_body,
        out_shape=jax.ShapeDtypeStruct((1, 1), jnp.float32),
    )(hist3, rowstats, scal, pred_labels, gt2, lw8)


def kernel(pred_labels, pred_mask, tgt_mask, gt_labels, loss_weight):
    hist = _run_sc_hist(pred_mask.reshape(NTOT), tgt_mask.reshape(NTOT))
    rowstats, scal = _run_pass1(pred_mask, tgt_mask)
    out = _run_combine(hist, rowstats, scal, pred_labels, gt_labels,
                       loss_weight)
    return out.reshape(())


# speedup vs baseline: 2.0722x; 2.0585x over previous
import functools

import jax
import jax.numpy as jnp
from jax import lax
from jax.experimental import pallas as pl
from jax.experimental.pallas import tpu as pltpu
from jax.experimental.pallas import tpu_sc as plsc

Q = 200
N = 50000
NTOT = Q * N
NCLS = 19
NON_OBJECT_WEIGHT = 0.1

NC = 2
NS = 16
NW = NC * NS

K = 2048
XLO = -8.0
XHI = 8.0
CHUNK = 10000
NCHUNKS = NTOT // CHUNK
ROWS_PER_STEP = 8
GRID1 = Q // ROWS_PER_STEP


def _pass1_body(x_ref, t_ref, rowstats_ref, scal_ref):
    i = pl.program_id(0)
    x = x_ref[...]
    t = t_ref[...]
    p = 1.0 / (1.0 + jnp.exp(-x))
    bce = jnp.maximum(x, 0.0) - x * t - jnp.log(jnp.maximum(p, 1.0 - p))
    bce_s = jnp.sum(bce)
    s_pt = jnp.sum(p * t, axis=1)
    s_p = jnp.sum(p, axis=1)
    s_t = jnp.sum(t, axis=1)
    col = lax.broadcasted_iota(jnp.int32, x.shape, 1).astype(jnp.float32)
    pm = p > 0.0
    gm = t > 0.0
    big = jnp.float32(N + 1)
    pmax_s = jnp.max(jnp.where(pm, col, -1.0))
    pmin_s = jnp.min(jnp.where(pm, col, big))
    gmax_s = jnp.max(jnp.where(gm, col, -1.0))
    gmin_s = jnp.min(jnp.where(gm, col, big))
    s1_s = jnp.sum(jnp.where(t > 0.5, 1.0 - p, 0.0))

    lane_r = lax.broadcasted_iota(jnp.int32, (ROWS_PER_STEP, 128), 1)
    rs = jnp.where(lane_r == 0, s_pt[:, None],
                   jnp.where(lane_r == 1, s_p[:, None],
                             jnp.where(lane_r == 2, s_t[:, None], 0.0)))
    rowstats_ref[...] = rs

    lane = lax.broadcasted_iota(jnp.int32, (1, 128), 1)
    v = jnp.where(lane == 0, bce_s,
        jnp.where(lane == 1, s1_s,
        jnp.where(lane == 2, pmax_s,
        jnp.where(lane == 3, pmin_s,
        jnp.where(lane == 4, gmax_s,
        jnp.where(lane == 5, gmin_s, 0.0))))))

    @pl.when(i == 0)
    def _init():
        scal_ref[...] = v

    @pl.when(i != 0)
    def _acc():
        old = scal_ref[...]
        upd = jnp.where(lane < 2, old + v,
              jnp.where((lane == 2) | (lane == 4), jnp.maximum(old, v),
              jnp.where((lane == 3) | (lane == 5), jnp.minimum(old, v), old)))
        scal_ref[...] = upd


def _run_pass1(pred_mask, tgt_mask):
    return pl.pallas_call(
        _pass1_body,
        grid=(GRID1,),
        in_specs=[
            pl.BlockSpec((ROWS_PER_STEP, N), lambda i: (i, 0)),
            pl.BlockSpec((ROWS_PER_STEP, N), lambda i: (i, 0)),
        ],
        out_specs=[
            pl.BlockSpec((ROWS_PER_STEP, 128), lambda i: (i, 0)),
            pl.BlockSpec((1, 128), lambda i: (0, 0)),
        ],
        out_shape=[
            jax.ShapeDtypeStruct((Q, 128), jnp.float32),
            jax.ShapeDtypeStruct((1, 128), jnp.float32),
        ],
    )(pred_mask, tgt_mask)


def _sc_hist_body(x_hbm, t_hbm, out_hbm,
                  xbuf0, tbuf0, xbuf1, tbuf1, hist, histb,
                  semx0, semt0, semx1, semt1):
    wid = lax.axis_index("s") * NC + lax.axis_index("c")
    zeros16 = jnp.zeros((16,), jnp.float32)
    ones16 = jnp.ones((16,), jnp.float32)
    lane16 = lax.iota(jnp.int32, 16)
    scale = jnp.float32(K / (XHI - XLO))
    off0 = jnp.float32(-XLO * K / (XHI - XLO))
    xbufs = (xbuf0, xbuf1)
    tbufs = (tbuf0, tbuf1)
    semxs = (semx0, semx1)
    semts = (semt0, semt1)

    @plsc.parallel_loop(0, (NS * K) // 16)
    def _zero(z):
        hist[pl.ds(z * 16, 16)] = zeros16
        histb[pl.ds(z * 16, 16)] = zeros16

    nchunks = jnp.where(wid < (NCHUNKS % NW), NCHUNKS // NW + 1, NCHUNKS // NW)

    def issue(ci, b):
        base = (wid + ci * NW) * CHUNK
        pltpu.async_copy(x_hbm.at[pl.ds(base, CHUNK)], xbufs[b], semxs[b])
        pltpu.async_copy(t_hbm.at[pl.ds(base, CHUNK)], tbufs[b], semts[b])

    issue(0, 0)
    npairs = (NCHUNKS // NW + 2) // 2

    @pl.loop(0, npairs)
    def _pair(pi):
        for b in (0, 1):
            ci = pi * 2 + b

            @pl.when(ci < nchunks)
            def _one():
                pltpu.make_async_copy(
                    x_hbm.at[pl.ds(0, CHUNK)], xbufs[b], semxs[b]).wait()
                pltpu.make_async_copy(
                    t_hbm.at[pl.ds(0, CHUNK)], tbufs[b], semts[b]).wait()

                @pl.when(ci + 1 < nchunks)
                def _prefetch():
                    issue(ci + 1, 1 - b)

                xbuf = xbufs[b]
                tbuf = tbufs[b]

                @plsc.parallel_loop(0, CHUNK // 128, unroll=2)
                def _vec(j8):
                    base8 = j8 * 128
                    for u in range(8):
                        xv = xbuf[pl.ds(base8 + u * 16, 16)]
                        tv = tbuf[pl.ds(base8 + u * 16, 16)]
                        binf = xv * scale + off0
                        bin_ = jnp.clip(binf.astype(jnp.int32), 0, K - 1)
                        idx = bin_ * 16 + lane16
                        plsc.addupdate_scatter(hist if u % 2 == 0 else histb,
                                               [idx], ones16,
                                               mask=tv < 0.5)

    @plsc.parallel_loop(0, (NS * K) // 16)
    def _merge(z):
        hist[pl.ds(z * 16, 16)] += histb[pl.ds(z * 16, 16)]

    pltpu.sync_copy(hist, out_hbm.at[pl.ds(wid * NS * K, NS * K)])


def _run_sc_hist(x_flat, t_flat):
    mesh = plsc.VectorSubcoreMesh(core_axis_name="c", subcore_axis_name="s")
    k = pl.kernel(
        _sc_hist_body,
        out_type=jax.ShapeDtypeStruct((NW * NS * K,), jnp.float32),
        mesh=mesh,
        scratch_types=[
            pltpu.VMEM((CHUNK,), jnp.float32),
            pltpu.VMEM((CHUNK,), jnp.float32),
            pltpu.VMEM((CHUNK,), jnp.float32),
            pltpu.VMEM((CHUNK,), jnp.float32),
            pltpu.VMEM((NS * K,), jnp.float32),
            pltpu.VMEM((NS * K,), jnp.float32),
            pltpu.SemaphoreType.DMA,
            pltpu.SemaphoreType.DMA,
            pltpu.SemaphoreType.DMA,
            pltpu.SemaphoreType.DMA,
        ],
        compiler_params=pltpu.CompilerParams(needs_layout_passes=False),
    )
    return k(x_flat, t_flat)


def _combine_body(hist_ref, rowstats_ref, scal_ref, pl_ref, gt_ref, lw_ref,
                  out_ref):
    scal = scal_ref[...]
    lane = lax.broadcasted_iota(jnp.int32, (1, 128), 1)

    def pick(j):
        return jnp.sum(jnp.where(lane == j, scal, 0.0))

    bce_sum = pick(0)
    s1 = pick(1)
    pmax = pick(2)
    pmin = pick(3)
    gmax = pick(4)
    gmin = pick(5)

    rs = rowstats_ref[...]
    s_pt = rs[:, 0:1]
    s_p = rs[:, 1:2]
    s_t = rs[:, 2:3]

    plog = pl_ref[...]
    g = gt_ref[...]
    m = jnp.max(plog, axis=1, keepdims=True)
    lse = m + jnp.log(jnp.sum(jnp.exp(plog - m), axis=1, keepdims=True))
    cls_iota = lax.broadcasted_iota(jnp.int32, (Q, NCLS), 1)
    onehot = (cls_iota == g).astype(jnp.float32)
    picked = jnp.sum(plog * onehot, axis=1, keepdims=True)
    ce = lse - picked
    w = jnp.where(g == NCLS - 1, NON_OBJECT_WEIGHT, 1.0).astype(jnp.float32)
    loss_class = jnp.sum(w * ce) / jnp.sum(w)

    n_f = jnp.float32(NTOT)
    loss_bce = bce_sum / n_f
    loss_dice = jnp.sum(1.0 - (2.0 * s_pt + 1.0) / (s_p + s_t + 1.0)) / Q
    eps = jnp.float32(1e-6)
    union = s_p + s_t - s_pt
    iou = s_pt / (union + eps)
    encl = (pmax - pmin) * (gmax - gmin)
    giou = iou - (encl - union) / (encl + eps)
    loss_giou = jnp.sum(1.0 - giou) / Q

    G = jnp.sum(s_t)
    n0 = n_f - G
    nrows = (K * 16) // 128
    h3 = hist_ref[...]
    h512 = jnp.sum(h3, axis=0)
    cc = lax.broadcasted_iota(jnp.int32, (128, 8), 0)
    ss = lax.broadcasted_iota(jnp.int32, (128, 8), 1)
    gather16 = ((cc >> 4) == ss).astype(jnp.float32)
    hb = lax.dot_general(h512, gather16, (((1,), (0,)), ((), ())),
                         preferred_element_type=jnp.float32)
    ui = lax.broadcasted_iota(jnp.int32, (8, 8), 0)
    uj = lax.broadcasted_iota(jnp.int32, (8, 8), 1)
    upper8 = (ui <= uj).astype(jnp.float32)
    c1 = lax.dot_general(hb, upper8, (((1,), (0,)), ((), ())),
                         preferred_element_type=jnp.float32)
    rt = c1[:, 7:8]
    si = lax.broadcasted_iota(jnp.int32, (nrows, nrows), 0)
    sj = lax.broadcasted_iota(jnp.int32, (nrows, nrows), 1)
    strictl = (sj < si).astype(jnp.float32)
    off = lax.dot_general(strictl, rt, (((1,), (0,)), ((), ())),
                          preferred_element_type=jnp.float32)
    S = c1 + off
    a = n0 - S
    b = a + hb
    kr = lax.broadcasted_iota(jnp.int32, (nrows, 8), 0).astype(jnp.float32)
    kc = lax.broadcasted_iota(jnp.int32, (nrows, 8), 1).astype(jnp.float32)
    kidx = kr * 8.0 + kc
    center = XLO + (kidx + 0.5) * ((XHI - XLO) / K)
    ebar = 1.0 + 1.0 / (1.0 + jnp.exp(-center))
    terms = ebar * G * hb / ((G + a) * (G + b))
    loss_lovasz = s1 / n_f + jnp.sum(terms)

    lw = lw_ref[...]
    lane8 = lax.broadcasted_iota(jnp.int32, (1, 8), 1)
    tv = jnp.where(lane8 == 0, loss_class,
         jnp.where(lane8 == 1, loss_bce,
         jnp.where(lane8 == 2, loss_dice,
         jnp.where(lane8 == 3, loss_giou,
         jnp.where(lane8 == 4, loss_lovasz, 0.0)))))
    out_ref[...] = jnp.sum(lw * tv).reshape(1, 1)


def _run_combine(hist, rowstats, scal, pred_labels, gt_labels, loss_weight):
    hist3 = hist.reshape(NW, (K * 16) // 128, 128)
    gt2 = gt_labels.astype(jnp.int32).reshape(Q, 1)
    lw8 = jnp.zeros((1, 8), jnp.float32).at[0, :5].set(loss_weight)
    return pl.pallas_call(
        _combine_body,
        out_shape=jax.ShapeDtypeStruct((1, 1), jnp.float32),
    )(hist3, rowstats, scal, pred_labels, gt2, lw8)


def kernel(pred_labels, pred_mask, tgt_mask, gt_labels, loss_weight):
    hist = _run_sc_hist(pred_mask.reshape(NTOT), tgt_mask.reshape(NTOT))
    rowstats, scal = _run_pass1(pred_mask, tgt_mask)
    out = _run_combine(hist, rowstats, scal, pred_labels, gt_labels,
                       loss_weight)
    return out.reshape(())


# speedup vs baseline: 2.3502x; 1.1341x over previous
import functools

import jax
import jax.numpy as jnp
from jax import lax
from jax.experimental import pallas as pl
from jax.experimental.pallas import tpu as pltpu
from jax.experimental.pallas import tpu_sc as plsc

Q = 200
N = 50000
NTOT = Q * N
NCLS = 19
NON_OBJECT_WEIGHT = 0.1

NC = 2
NS = 16
NW = NC * NS

K = 2048
XLO = -8.0
XHI = 8.0
CHUNK = 20000
NCHUNKS = NTOT // CHUNK
ROWS_PER_STEP = 8
GRID1 = Q // ROWS_PER_STEP


def _pass1_body(x_ref, t_ref, rowstats_ref, scal_ref):
    i = pl.program_id(0)
    x = x_ref[...]
    t = t_ref[...]
    p = 1.0 / (1.0 + jnp.exp(-x))
    bce = jnp.maximum(x, 0.0) - x * t - jnp.log(jnp.maximum(p, 1.0 - p))
    bce_s = jnp.sum(bce)
    s_pt = jnp.sum(p * t, axis=1)
    s_p = jnp.sum(p, axis=1)
    s_t = jnp.sum(t, axis=1)
    col = lax.broadcasted_iota(jnp.int32, x.shape, 1).astype(jnp.float32)
    pm = p > 0.0
    gm = t > 0.0
    big = jnp.float32(N + 1)
    pmax_s = jnp.max(jnp.where(pm, col, -1.0))
    pmin_s = jnp.min(jnp.where(pm, col, big))
    gmax_s = jnp.max(jnp.where(gm, col, -1.0))
    gmin_s = jnp.min(jnp.where(gm, col, big))
    s1_s = jnp.sum(jnp.where(t > 0.5, 1.0 - p, 0.0))

    lane_r = lax.broadcasted_iota(jnp.int32, (ROWS_PER_STEP, 128), 1)
    rs = jnp.where(lane_r == 0, s_pt[:, None],
                   jnp.where(lane_r == 1, s_p[:, None],
                             jnp.where(lane_r == 2, s_t[:, None], 0.0)))
    rowstats_ref[...] = rs

    lane = lax.broadcasted_iota(jnp.int32, (1, 128), 1)
    v = jnp.where(lane == 0, bce_s,
        jnp.where(lane == 1, s1_s,
        jnp.where(lane == 2, pmax_s,
        jnp.where(lane == 3, pmin_s,
        jnp.where(lane == 4, gmax_s,
        jnp.where(lane == 5, gmin_s, 0.0))))))

    @pl.when(i == 0)
    def _init():
        scal_ref[...] = v

    @pl.when(i != 0)
    def _acc():
        old = scal_ref[...]
        upd = jnp.where(lane < 2, old + v,
              jnp.where((lane == 2) | (lane == 4), jnp.maximum(old, v),
              jnp.where((lane == 3) | (lane == 5), jnp.minimum(old, v), old)))
        scal_ref[...] = upd


def _run_pass1(pred_mask, tgt_mask):
    return pl.pallas_call(
        _pass1_body,
        grid=(GRID1,),
        in_specs=[
            pl.BlockSpec((ROWS_PER_STEP, N), lambda i: (i, 0)),
            pl.BlockSpec((ROWS_PER_STEP, N), lambda i: (i, 0)),
        ],
        out_specs=[
            pl.BlockSpec((ROWS_PER_STEP, 128), lambda i: (i, 0)),
            pl.BlockSpec((1, 128), lambda i: (0, 0)),
        ],
        out_shape=[
            jax.ShapeDtypeStruct((Q, 128), jnp.float32),
            jax.ShapeDtypeStruct((1, 128), jnp.float32),
        ],
    )(pred_mask, tgt_mask)


def _sc_hist_body(x_hbm, out_hbm,
                  xbuf0, xbuf1, hist, histb,
                  semx0, semx1):
    wid = lax.axis_index("s") * NC + lax.axis_index("c")
    zeros16 = jnp.zeros((16,), jnp.float32)
    ones16 = jnp.ones((16,), jnp.float32)
    lane16 = lax.iota(jnp.int32, 16)
    scale = jnp.float32(K / (XHI - XLO))
    off0 = jnp.float32(-XLO * K / (XHI - XLO))
    xbufs = (xbuf0, xbuf1)
    semxs = (semx0, semx1)

    @plsc.parallel_loop(0, (NS * K) // 16)
    def _zero(z):
        hist[pl.ds(z * 16, 16)] = zeros16
        histb[pl.ds(z * 16, 16)] = zeros16

    nchunks = jnp.where(wid < (NCHUNKS % NW), NCHUNKS // NW + 1, NCHUNKS // NW)

    def issue(ci, b):
        base = (wid + ci * NW) * CHUNK
        pltpu.async_copy(x_hbm.at[pl.ds(base, CHUNK)], xbufs[b], semxs[b])

    issue(0, 0)
    npairs = (NCHUNKS // NW + 2) // 2

    @pl.loop(0, npairs)
    def _pair(pi):
        for b in (0, 1):
            ci = pi * 2 + b

            @pl.when(ci < nchunks)
            def _one():
                pltpu.make_async_copy(
                    x_hbm.at[pl.ds(0, CHUNK)], xbufs[b], semxs[b]).wait()

                @pl.when(ci + 1 < nchunks)
                def _prefetch():
                    issue(ci + 1, 1 - b)

                xbuf = xbufs[b]

                @plsc.parallel_loop(0, CHUNK // 128, unroll=2)
                def _vec(j8):
                    base8 = j8 * 128
                    for u in range(8):
                        xv = xbuf[pl.ds(base8 + u * 16, 16)]
                        binf = jnp.clip(xv * scale + off0, 0.0,
                                        jnp.float32(K - 1))
                        bin_ = binf.astype(jnp.int32)
                        idx = bin_ * 16 + lane16
                        plsc.addupdate_scatter(hist if u % 2 == 0 else histb,
                                               [idx], ones16)

    @plsc.parallel_loop(0, (NS * K) // 16)
    def _merge(z):
        hist[pl.ds(z * 16, 16)] += histb[pl.ds(z * 16, 16)]

    pltpu.sync_copy(hist, out_hbm.at[pl.ds(wid * NS * K, NS * K)])


def _run_sc_hist(x_flat):
    mesh = plsc.VectorSubcoreMesh(core_axis_name="c", subcore_axis_name="s")
    k = pl.kernel(
        _sc_hist_body,
        out_type=jax.ShapeDtypeStruct((NW * NS * K,), jnp.float32),
        mesh=mesh,
        scratch_types=[
            pltpu.VMEM((CHUNK,), jnp.float32),
            pltpu.VMEM((CHUNK,), jnp.float32),
            pltpu.VMEM((NS * K,), jnp.float32),
            pltpu.VMEM((NS * K,), jnp.float32),
            pltpu.SemaphoreType.DMA,
            pltpu.SemaphoreType.DMA,
        ],
        compiler_params=pltpu.CompilerParams(needs_layout_passes=False),
    )
    return k(x_flat)


def _combine_body(hist_ref, rowstats_ref, scal_ref, pl_ref, gt_ref, lw_ref,
                  out_ref):
    scal = scal_ref[...]
    lane = lax.broadcasted_iota(jnp.int32, (1, 128), 1)

    def pick(j):
        return jnp.sum(jnp.where(lane == j, scal, 0.0))

    bce_sum = pick(0)
    s1 = pick(1)
    pmax = pick(2)
    pmin = pick(3)
    gmax = pick(4)
    gmin = pick(5)

    rs = rowstats_ref[...]
    s_pt = rs[:, 0:1]
    s_p = rs[:, 1:2]
    s_t = rs[:, 2:3]

    plog = pl_ref[...]
    g = gt_ref[...]
    m = jnp.max(plog, axis=1, keepdims=True)
    lse = m + jnp.log(jnp.sum(jnp.exp(plog - m), axis=1, keepdims=True))
    cls_iota = lax.broadcasted_iota(jnp.int32, (Q, NCLS), 1)
    onehot = (cls_iota == g).astype(jnp.float32)
    picked = jnp.sum(plog * onehot, axis=1, keepdims=True)
    ce = lse - picked
    w = jnp.where(g == NCLS - 1, NON_OBJECT_WEIGHT, 1.0).astype(jnp.float32)
    loss_class = jnp.sum(w * ce) / jnp.sum(w)

    n_f = jnp.float32(NTOT)
    loss_bce = bce_sum / n_f
    loss_dice = jnp.sum(1.0 - (2.0 * s_pt + 1.0) / (s_p + s_t + 1.0)) / Q
    eps = jnp.float32(1e-6)
    union = s_p + s_t - s_pt
    iou = s_pt / (union + eps)
    encl = (pmax - pmin) * (gmax - gmin)
    giou = iou - (encl - union) / (encl + eps)
    loss_giou = jnp.sum(1.0 - giou) / Q

    G = jnp.sum(s_t)
    n0 = n_f - G
    nrows = (K * 16) // 128
    h3 = hist_ref[...]
    h512 = jnp.sum(h3, axis=0)
    cc = lax.broadcasted_iota(jnp.int32, (128, 8), 0)
    ss = lax.broadcasted_iota(jnp.int32, (128, 8), 1)
    gather16 = ((cc >> 4) == ss).astype(jnp.float32)
    hb = lax.dot_general(h512, gather16, (((1,), (0,)), ((), ())),
                         preferred_element_type=jnp.float32)
    rz = lax.broadcasted_iota(jnp.int32, hb.shape, 0)
    cz = lax.broadcasted_iota(jnp.int32, hb.shape, 1)
    hb = hb - jnp.where((rz == 0) & (cz == 0), G, 0.0)
    ui = lax.broadcasted_iota(jnp.int32, (8, 8), 0)
    uj = lax.broadcasted_iota(jnp.int32, (8, 8), 1)
    upper8 = (ui <= uj).astype(jnp.float32)
    c1 = lax.dot_general(hb, upper8, (((1,), (0,)), ((), ())),
                         preferred_element_type=jnp.float32)
    rt = c1[:, 7:8]
    si = lax.broadcasted_iota(jnp.int32, (nrows, nrows), 0)
    sj = lax.broadcasted_iota(jnp.int32, (nrows, nrows), 1)
    strictl = (sj < si).astype(jnp.float32)
    off = lax.dot_general(strictl, rt, (((1,), (0,)), ((), ())),
                          preferred_element_type=jnp.float32)
    S = c1 + off
    a = n0 - S
    b = a + hb
    kr = lax.broadcasted_iota(jnp.int32, (nrows, 8), 0).astype(jnp.float32)
    kc = lax.broadcasted_iota(jnp.int32, (nrows, 8), 1).astype(jnp.float32)
    kidx = kr * 8.0 + kc
    center = XLO + (kidx + 0.5) * ((XHI - XLO) / K)
    ebar = 1.0 + 1.0 / (1.0 + jnp.exp(-center))
    terms = ebar * G * hb / ((G + a) * (G + b))
    loss_lovasz = s1 / n_f + jnp.sum(terms)

    lw = lw_ref[...]
    lane8 = lax.broadcasted_iota(jnp.int32, (1, 8), 1)
    tv = jnp.where(lane8 == 0, loss_class,
         jnp.where(lane8 == 1, loss_bce,
         jnp.where(lane8 == 2, loss_dice,
         jnp.where(lane8 == 3, loss_giou,
         jnp.where(lane8 == 4, loss_lovasz, 0.0)))))
    out_ref[...] = jnp.sum(lw * tv).reshape(1, 1)


def _run_combine(hist, rowstats, scal, pred_labels, gt_labels, loss_weight):
    hist3 = hist.reshape(NW, (K * 16) // 128, 128)
    gt2 = gt_labels.astype(jnp.int32).reshape(Q, 1)
    lw8 = jnp.zeros((1, 8), jnp.float32).at[0, :5].set(loss_weight)
    return pl.pallas_call(
        _combine_body,
        out_shape=jax.ShapeDtypeStruct((1, 1), jnp.float32),
    )(hist3, rowstats, scal, pred_labels, gt2, lw8)


def kernel(pred_labels, pred_mask, tgt_mask, gt_labels, loss_weight):
    fused = jnp.where(tgt_mask > 0.5, jnp.float32(-1e30), pred_mask)
    hist = _run_sc_hist(fused.reshape(NTOT))
    rowstats, scal = _run_pass1(pred_mask, tgt_mask)
    out = _run_combine(hist, rowstats, scal, pred_labels, gt_labels,
                       loss_weight)
    return out.reshape(())
